# Initial kernel scaffold; baseline (speedup 1.0000x reference)
import functools

import jax
import jax.numpy as jnp
from jax import lax
from jax.experimental import pallas as pl
from jax.experimental.pallas import tpu as pltpu
from jax.experimental.pallas import tpu_sc as plsc

N = 10000
E = 160000
D = 256
H = 128
NC = 2
NS = 16
RPT = N // NS

EP1 = E // (NC * NS)
K1 = 40
NCH1 = EP1 // K1

EP2 = E // NS
K2 = 80
NCH2 = EP2 // K2

ZR = 125

_mesh = plsc.VectorSubcoreMesh(core_axis_name="c", subcore_axis_name="s")


@jax.jit
def _deg_sc(row):
    @functools.partial(
        pl.kernel,
        out_type=[
            jax.ShapeDtypeStruct((N, 16), jnp.float32),
            jax.ShapeDtypeStruct((N, 16), jnp.float32),
        ],
        mesh=_mesh,
        scratch_types=[
            pltpu.VMEM_SHARED((N, 16), jnp.float32),
            pltpu.VMEM((K1,), jnp.int32),
            pltpu.VMEM((K1, 16), jnp.float32),
            pltpu.VMEM((RPT, 16), jnp.float32),
        ],
    )
    def k(row_hbm, dega_hbm, degb_hbm, acc_sh, idx_v, ones_v, zbuf_v):
        c = lax.axis_index("c")
        s = lax.axis_index("s")

        @pl.loop(0, RPT)
        def _(i):
            zbuf_v[pl.ds(i, 1), :] = jnp.zeros((1, 16), jnp.float32)

        @pl.loop(0, K1)
        def _(i):
            ones_v[pl.ds(i, 1), :] = jnp.ones((1, 16), jnp.float32)

        pltpu.sync_copy(zbuf_v, acc_sh.at[pl.ds(s * RPT, RPT)])
        plsc.subcore_barrier()

        base = (s * NC + c) * EP1

        @pl.loop(0, NCH1)
        def _(i):
            pltpu.sync_copy(row_hbm.at[pl.ds(base + i * K1, K1)], idx_v)
            pltpu.sync_copy(ones_v, acc_sh.at[idx_v], add=True)

        plsc.subcore_barrier()
        sl = pl.ds(s * RPT, RPT)

        @pl.when(c == 0)
        def _():
            pltpu.sync_copy(acc_sh.at[sl], dega_hbm.at[sl])

        @pl.when(c == 1)
        def _():
            pltpu.sync_copy(acc_sh.at[sl], degb_hbm.at[sl])

    return k(row)


_RB = 1000


def _dinv_block(da, db):
    deg = da + db
    dinv = jnp.where(deg > 0, lax.rsqrt(jnp.maximum(deg, 1.0)), 0.0)
    return dinv[:, 0:1]


def _prescale_body(x_ref, da_ref, db_ref, oa_ref, ob_ref):
    d1 = _dinv_block(da_ref[...], db_ref[...])
    xs = x_ref[...] * (-d1)
    oa_ref[...] = xs[:, :H]
    ob_ref[...] = xs[:, H:]


@jax.jit
def _prescale(x, dega, degb):
    grid = (N // _RB,)
    return pl.pallas_call(
        _prescale_body,
        grid=grid,
        in_specs=[
            pl.BlockSpec((_RB, D), lambda i: (i, 0)),
            pl.BlockSpec((_RB, 16), lambda i: (i, 0)),
            pl.BlockSpec((_RB, 16), lambda i: (i, 0)),
        ],
        out_specs=[
            pl.BlockSpec((_RB, H), lambda i: (i, 0)),
            pl.BlockSpec((_RB, H), lambda i: (i, 0)),
        ],
        out_shape=[
            jax.ShapeDtypeStruct((N, H), jnp.float32),
            jax.ShapeDtypeStruct((N, H), jnp.float32),
        ],
    )(x, dega, degb)


@jax.jit
def _scatter_sc(row, col, xsa, xsb):
    @functools.partial(
        pl.kernel,
        out_type=[
            jax.ShapeDtypeStruct((N, H), jnp.float32),
            jax.ShapeDtypeStruct((N, H), jnp.float32),
        ],
        mesh=_mesh,
        scratch_types=[
            pltpu.VMEM_SHARED((N, H), jnp.float32),
            pltpu.VMEM((K2,), jnp.int32),
            pltpu.VMEM((K2,), jnp.int32),
            pltpu.VMEM((K2, H), jnp.float32),
            pltpu.VMEM((ZR, H), jnp.float32),
            pltpu.SemaphoreType.DMA,
        ],
    )
    def k(row_hbm, col_hbm, xsa_hbm, xsb_hbm, txa_hbm, txb_hbm,
          acc_sh, ridx_v, cidx_v, rows_v, zbuf_v, sem):
        c = lax.axis_index("c")
        s = lax.axis_index("s")

        @pl.loop(0, ZR)
        def _(i):
            @pl.loop(0, H // 16)
            def _(j):
                zbuf_v[pl.ds(i, 1), pl.ds(j * 16, 16)] = (
                    jnp.zeros((1, 16), jnp.float32))

        @pl.loop(0, RPT // ZR)
        def _(j):
            pltpu.sync_copy(zbuf_v, acc_sh.at[pl.ds(s * RPT + j * ZR, ZR)])

        plsc.subcore_barrier()

        base = s * EP2

        def run(xs_hbm):
            @pl.loop(0, NCH2)
            def _(i):
                off = pl.ds(base + i * K2, K2)
                pltpu.sync_copy(row_hbm.at[off], ridx_v)
                pltpu.sync_copy(col_hbm.at[off], cidx_v)
                pltpu.async_copy(xs_hbm.at[ridx_v], rows_v, sem).wait()
                pltpu.sync_copy(rows_v, acc_sh.at[cidx_v], add=True)

        @pl.when(c == 0)
        def _():
            run(xsa_hbm)

        @pl.when(c == 1)
        def _():
            run(xsb_hbm)

        plsc.subcore_barrier()
        sl = pl.ds(s * RPT, RPT)

        @pl.when(c == 0)
        def _():
            pltpu.sync_copy(acc_sh.at[sl], txa_hbm.at[sl])

        @pl.when(c == 1)
        def _():
            pltpu.sync_copy(acc_sh.at[sl], txb_hbm.at[sl])

    return k(row, col, xsa, xsb)


def _final_body(x_ref, ta_ref, tb_ref, da_ref, db_ref,
                w0_ref, w1_ref, b_ref, o_ref):
    d1 = _dinv_block(da_ref[...], db_ref[...])
    acc = jnp.dot(x_ref[...], w0_ref[...],
                  preferred_element_type=jnp.float32)
    acc += jnp.dot(ta_ref[...] * d1, w1_ref[:H, :],
                   preferred_element_type=jnp.float32)
    acc += jnp.dot(tb_ref[...] * d1, w1_ref[H:, :],
                   preferred_element_type=jnp.float32)
    o_ref[...] = jnp.maximum(acc + b_ref[...], 0.0)


@jax.jit
def _final(x, txa, txb, dega, degb, W0, W1, b2):
    grid = (N // _RB,)
    return pl.pallas_call(
        _final_body,
        grid=grid,
        in_specs=[
            pl.BlockSpec((_RB, D), lambda i: (i, 0)),
            pl.BlockSpec((_RB, H), lambda i: (i, 0)),
            pl.BlockSpec((_RB, H), lambda i: (i, 0)),
            pl.BlockSpec((_RB, 16), lambda i: (i, 0)),
            pl.BlockSpec((_RB, 16), lambda i: (i, 0)),
            pl.BlockSpec((D, D), lambda i: (0, 0)),
            pl.BlockSpec((D, D), lambda i: (0, 0)),
            pl.BlockSpec((1, D), lambda i: (0, 0)),
        ],
        out_specs=pl.BlockSpec((_RB, D), lambda i: (i, 0)),
        out_shape=jax.ShapeDtypeStruct((N, D), jnp.float32),
    )(x, txa, txb, dega, degb, W0, W1, b2)


def kernel(x, edge_index, W0, W1, b):
    row = edge_index[0]
    col = edge_index[1]
    dega, degb = _deg_sc(row)
    xsa, xsb = _prescale(x, dega, degb)
    txa, txb = _scatter_sc(row, col, xsa, xsb)
    return _final(x, txa, txb, dega, degb, W0, W1, b.reshape(1, D))


# speedup vs baseline: 7.3318x; 7.3318x over previous
import functools

import jax
import jax.numpy as jnp
from jax import lax
from jax.experimental import pallas as pl
from jax.experimental.pallas import tpu as pltpu
from jax.experimental.pallas import tpu_sc as plsc

N = 10000
E = 160000
D = 256
H = 128
NC = 2
NS = 16
RA = 624
TAIL_OFF = RA * NS
TAIL = N - TAIL_OFF

EP1 = E // (NC * NS)
K1 = 40
NCH1 = EP1 // K1

EP2 = E // NS
K2 = 80
NCH2 = EP2 // K2

_mesh = plsc.VectorSubcoreMesh(core_axis_name="c", subcore_axis_name="s")


@jax.jit
def _deg_sc(row2, ones, zeros):
    @functools.partial(
        pl.kernel,
        out_type=[
            jax.ShapeDtypeStruct((N, H), jnp.float32),
            jax.ShapeDtypeStruct((N, H), jnp.float32),
        ],
        mesh=_mesh,
        scratch_types=[
            pltpu.VMEM_SHARED((N, H), jnp.float32),
            pltpu.VMEM((K1,), jnp.int32),
            pltpu.VMEM((K1, H), jnp.float32),
        ],
    )
    def k(row_hbm, ones_hbm, zeros_hbm, dega_hbm, degb_hbm,
          acc_sh, idx_v, ones_v):
        c = lax.axis_index("c")
        s = lax.axis_index("s")

        pltpu.sync_copy(ones_hbm, ones_v)
        sl = pl.ds(s * RA, RA)
        tl = pl.ds(TAIL_OFF, TAIL)
        pltpu.sync_copy(zeros_hbm.at[sl], acc_sh.at[sl])

        @pl.when(s == NS - 1)
        def _():
            pltpu.sync_copy(zeros_hbm.at[tl], acc_sh.at[tl])

        plsc.subcore_barrier()

        base = (s * NC + c) * EP1

        @pl.loop(0, NCH1)
        def _(i):
            pltpu.sync_copy(row_hbm.at[pl.ds(base + i * K1, K1)], idx_v)
            pltpu.sync_copy(ones_v, acc_sh.at[idx_v], add=True)

        plsc.subcore_barrier()

        @pl.when(c == 0)
        def _():
            pltpu.sync_copy(acc_sh.at[sl], dega_hbm.at[sl])

            @pl.when(s == NS - 1)
            def _():
                pltpu.sync_copy(acc_sh.at[tl], dega_hbm.at[tl])

        @pl.when(c == 1)
        def _():
            pltpu.sync_copy(acc_sh.at[sl], degb_hbm.at[sl])

            @pl.when(s == NS - 1)
            def _():
                pltpu.sync_copy(acc_sh.at[tl], degb_hbm.at[tl])

    return k(row2, ones, zeros)


_RB = 1000


def _dinv_block(da, db):
    deg = da + db
    return jnp.where(deg > 0, lax.rsqrt(jnp.maximum(deg, 1.0)), 0.0)


def _prescale_body(x_ref, da_ref, db_ref, oa_ref, ob_ref):
    d1 = _dinv_block(da_ref[:, 0:1], db_ref[:, 0:1])
    xs = x_ref[...] * (-d1)
    oa_ref[...] = xs[:, :H]
    ob_ref[...] = xs[:, H:]


@jax.jit
def _prescale(x, dega, degb):
    grid = (N // _RB,)
    return pl.pallas_call(
        _prescale_body,
        grid=grid,
        in_specs=[
            pl.BlockSpec((_RB, D), lambda i: (i, 0)),
            pl.BlockSpec((_RB, H), lambda i: (i, 0)),
            pl.BlockSpec((_RB, H), lambda i: (i, 0)),
        ],
        out_specs=[
            pl.BlockSpec((_RB, H), lambda i: (i, 0)),
            pl.BlockSpec((_RB, H), lambda i: (i, 0)),
        ],
        out_shape=[
            jax.ShapeDtypeStruct((N, H), jnp.float32),
            jax.ShapeDtypeStruct((N, H), jnp.float32),
        ],
    )(x, dega, degb)


@jax.jit
def _scatter_sc(row2, col2, xsa, xsb, zeros):
    @functools.partial(
        pl.kernel,
        out_type=[
            jax.ShapeDtypeStruct((N, H), jnp.float32),
            jax.ShapeDtypeStruct((N, H), jnp.float32),
        ],
        mesh=_mesh,
        scratch_types=[
            pltpu.VMEM_SHARED((N, H), jnp.float32),
            pltpu.VMEM((K2,), jnp.int32),
            pltpu.VMEM((K2,), jnp.int32),
            pltpu.VMEM((K2, H), jnp.float32),
            pltpu.SemaphoreType.DMA,
        ],
    )
    def k(row_hbm, col_hbm, xsa_hbm, xsb_hbm, zeros_hbm, txa_hbm, txb_hbm,
          acc_sh, ridx_v, cidx_v, rows_v, sem):
        c = lax.axis_index("c")
        s = lax.axis_index("s")

        sl = pl.ds(s * RA, RA)
        tl = pl.ds(TAIL_OFF, TAIL)
        pltpu.sync_copy(zeros_hbm.at[sl], acc_sh.at[sl])

        @pl.when(s == NS - 1)
        def _():
            pltpu.sync_copy(zeros_hbm.at[tl], acc_sh.at[tl])

        plsc.subcore_barrier()

        base = s * EP2

        def run(xs_hbm):
            @pl.loop(0, NCH2)
            def _(i):
                off = pl.ds(base + i * K2, K2)
                pltpu.sync_copy(row_hbm.at[off], ridx_v)
                pltpu.sync_copy(col_hbm.at[off], cidx_v)
                pltpu.async_copy(xs_hbm.at[ridx_v], rows_v, sem).wait()
                pltpu.sync_copy(rows_v, acc_sh.at[cidx_v], add=True)

        @pl.when(c == 0)
        def _():
            run(xsa_hbm)

        @pl.when(c == 1)
        def _():
            run(xsb_hbm)

        plsc.subcore_barrier()

        @pl.when(c == 0)
        def _():
            pltpu.sync_copy(acc_sh.at[sl], txa_hbm.at[sl])

            @pl.when(s == NS - 1)
            def _():
                pltpu.sync_copy(acc_sh.at[tl], txa_hbm.at[tl])

        @pl.when(c == 1)
        def _():
            pltpu.sync_copy(acc_sh.at[sl], txb_hbm.at[sl])

            @pl.when(s == NS - 1)
            def _():
                pltpu.sync_copy(acc_sh.at[tl], txb_hbm.at[tl])

    return k(row2, col2, xsa, xsb, zeros)


def _xw0_body(x_ref, w0_ref, b_ref, o_ref):
    o_ref[...] = jnp.dot(x_ref[...], w0_ref[...],
                         preferred_element_type=jnp.float32) + b_ref[...]


@jax.jit
def _xw0(x, W0, b2):
    grid = (N // _RB,)
    return pl.pallas_call(
        _xw0_body,
        grid=grid,
        in_specs=[
            pl.BlockSpec((_RB, D), lambda i: (i, 0)),
            pl.BlockSpec((D, D), lambda i: (0, 0)),
            pl.BlockSpec((1, D), lambda i: (0, 0)),
        ],
        out_specs=pl.BlockSpec((_RB, D), lambda i: (i, 0)),
        out_shape=jax.ShapeDtypeStruct((N, D), jnp.float32),
    )(x, W0, b2)


def _final_body(p_ref, ta_ref, tb_ref, da_ref, db_ref, w1_ref, o_ref):
    d1 = _dinv_block(da_ref[:, 0:1], db_ref[:, 0:1])
    acc = p_ref[...]
    acc += jnp.dot(ta_ref[...] * d1, w1_ref[:H, :],
                   preferred_element_type=jnp.float32)
    acc += jnp.dot(tb_ref[...] * d1, w1_ref[H:, :],
                   preferred_element_type=jnp.float32)
    o_ref[...] = jnp.maximum(acc, 0.0)


@jax.jit
def _final(p, txa, txb, dega, degb, W1):
    grid = (N // _RB,)
    return pl.pallas_call(
        _final_body,
        grid=grid,
        in_specs=[
            pl.BlockSpec((_RB, D), lambda i: (i, 0)),
            pl.BlockSpec((_RB, H), lambda i: (i, 0)),
            pl.BlockSpec((_RB, H), lambda i: (i, 0)),
            pl.BlockSpec((_RB, H), lambda i: (i, 0)),
            pl.BlockSpec((_RB, H), lambda i: (i, 0)),
            pl.BlockSpec((D, D), lambda i: (0, 0)),
        ],
        out_specs=pl.BlockSpec((_RB, D), lambda i: (i, 0)),
        out_shape=jax.ShapeDtypeStruct((N, D), jnp.float32),
    )(p, txa, txb, dega, degb, W1)


def kernel(x, edge_index, W0, W1, b):
    row2 = edge_index[0]
    col2 = edge_index[1]
    ones = jnp.ones((K1, H), jnp.float32)
    zeros = jnp.zeros((N, H), jnp.float32)
    p = _xw0(x, W0, b.reshape(1, D))
    dega, degb = _deg_sc(row2, ones, zeros)
    xsa, xsb = _prescale(x, dega, degb)
    txa, txb = _scatter_sc(row2, col2, xsa, xsb, zeros)
    return _final(p, txa, txb, dega, degb, W1)


# speedup vs baseline: 12.3537x; 1.6850x over previous
import functools

import jax
import jax.numpy as jnp
from jax import lax
from jax.experimental import pallas as pl
from jax.experimental.pallas import tpu as pltpu
from jax.experimental.pallas import tpu_sc as plsc

N = 10000
E = 160000
D = 256
H = 128
NC = 2
NS = 16
RA = 624
TAIL_OFF = RA * NS
TAIL = N - TAIL_OFF

EP1 = E // (NC * NS)
K1 = 40
NCH1 = EP1 // K1

EP2 = E // NS
K2 = 80
NCH2 = EP2 // K2

_mesh = plsc.VectorSubcoreMesh(core_axis_name="c", subcore_axis_name="s")


@jax.jit
def _deg_sc(row2, ones, zeros):
    @functools.partial(
        pl.kernel,
        out_type=[
            jax.ShapeDtypeStruct((N, H), jnp.float32),
            jax.ShapeDtypeStruct((N, H), jnp.float32),
        ],
        mesh=_mesh,
        scratch_types=[
            pltpu.VMEM_SHARED((N, H), jnp.float32),
            pltpu.VMEM((K1,), jnp.int32),
            pltpu.VMEM((K1, H), jnp.float32),
        ],
    )
    def k(row_hbm, ones_hbm, zeros_hbm, dega_hbm, degb_hbm,
          acc_sh, idx_v, ones_v):
        c = lax.axis_index("c")
        s = lax.axis_index("s")

        pltpu.sync_copy(ones_hbm, ones_v)
        sl = pl.ds(s * RA, RA)
        tl = pl.ds(TAIL_OFF, TAIL)
        pltpu.sync_copy(zeros_hbm.at[sl], acc_sh.at[sl])

        @pl.when(s == NS - 1)
        def _():
            pltpu.sync_copy(zeros_hbm.at[tl], acc_sh.at[tl])

        plsc.subcore_barrier()

        base = (s * NC + c) * EP1

        @pl.loop(0, NCH1)
        def _(i):
            pltpu.sync_copy(row_hbm.at[pl.ds(base + i * K1, K1)], idx_v)
            pltpu.sync_copy(ones_v, acc_sh.at[idx_v], add=True)

        plsc.subcore_barrier()

        @pl.when(c == 0)
        def _():
            pltpu.sync_copy(acc_sh.at[sl], dega_hbm.at[sl])

            @pl.when(s == NS - 1)
            def _():
                pltpu.sync_copy(acc_sh.at[tl], dega_hbm.at[tl])

        @pl.when(c == 1)
        def _():
            pltpu.sync_copy(acc_sh.at[sl], degb_hbm.at[sl])

            @pl.when(s == NS - 1)
            def _():
                pltpu.sync_copy(acc_sh.at[tl], degb_hbm.at[tl])

    return k(row2, ones, zeros)


_RB = 1000


def _dinv_block(da, db):
    deg = da + db
    return jnp.where(deg > 0, lax.rsqrt(jnp.maximum(deg, 1.0)), 0.0)


def _prescale_body(x_ref, da_ref, db_ref, oa_ref, ob_ref):
    d1 = _dinv_block(da_ref[:, 0:1], db_ref[:, 0:1])
    xs = x_ref[...] * (-d1)
    oa_ref[...] = xs[:, :H]
    ob_ref[...] = xs[:, H:]


@jax.jit
def _prescale(x, dega, degb):
    grid = (N // _RB,)
    return pl.pallas_call(
        _prescale_body,
        grid=grid,
        in_specs=[
            pl.BlockSpec((_RB, D), lambda i: (i, 0)),
            pl.BlockSpec((_RB, H), lambda i: (i, 0)),
            pl.BlockSpec((_RB, H), lambda i: (i, 0)),
        ],
        out_specs=[
            pl.BlockSpec((_RB, H), lambda i: (i, 0)),
            pl.BlockSpec((_RB, H), lambda i: (i, 0)),
        ],
        out_shape=[
            jax.ShapeDtypeStruct((N, H), jnp.float32),
            jax.ShapeDtypeStruct((N, H), jnp.float32),
        ],
    )(x, dega, degb)


@jax.jit
def _scatter_sc(row2, col2, xsa, xsb, zeros):
    @functools.partial(
        pl.kernel,
        out_type=[
            jax.ShapeDtypeStruct((N, H), jnp.float32),
            jax.ShapeDtypeStruct((N, H), jnp.float32),
        ],
        mesh=_mesh,
        scratch_types=[
            pltpu.VMEM_SHARED((N, H), jnp.float32),
            pltpu.VMEM((EP2,), jnp.int32),
            pltpu.VMEM((K2,), jnp.int32),
            pltpu.VMEM((K2,), jnp.int32),
            pltpu.VMEM((K2, H), jnp.float32),
            pltpu.VMEM((K2, H), jnp.float32),
            pltpu.SemaphoreType.DMA,
            pltpu.SemaphoreType.DMA,
            pltpu.SemaphoreType.DMA,
            pltpu.SemaphoreType.DMA,
        ],
    )
    def k(row_hbm, col_hbm, xsa_hbm, xsb_hbm, zeros_hbm, txa_hbm, txb_hbm,
          acc_sh, ridx_v, cidx0_v, cidx1_v, rows0_v, rows1_v,
          gsem0, gsem1, csem0, csem1):
        c = lax.axis_index("c")
        s = lax.axis_index("s")

        sl = pl.ds(s * RA, RA)
        tl = pl.ds(TAIL_OFF, TAIL)
        pltpu.sync_copy(zeros_hbm.at[sl], acc_sh.at[sl])

        @pl.when(s == NS - 1)
        def _():
            pltpu.sync_copy(zeros_hbm.at[tl], acc_sh.at[tl])

        plsc.subcore_barrier()

        base = s * EP2
        cidx = (cidx0_v, cidx1_v)
        rows = (rows0_v, rows1_v)
        gsem = (gsem0, gsem1)
        csem = (csem0, csem1)

        def run(xs_hbm):
            pltpu.sync_copy(row_hbm.at[pl.ds(base, EP2)], ridx_v)

            def start(i, b):
                pltpu.async_copy(col_hbm.at[pl.ds(base + i * K2, K2)],
                                 cidx[b], csem[b])
                pltpu.async_copy(
                    xs_hbm.at[ridx_v.at[pl.ds(i * K2, K2)]], rows[b], gsem[b])

            def finish(i, b):
                pltpu.make_async_copy(col_hbm.at[pl.ds(base, K2)],
                                      cidx[b], csem[b]).wait()
                pltpu.make_async_copy(zeros_hbm.at[pl.ds(0, K2)], rows[b],
                                      gsem[b]).wait()
                pltpu.sync_copy(rows[b], acc_sh.at[cidx[b]], add=True)

            start(0, 0)

            @pl.loop(0, (NCH2 - 1) // 2)
            def _(j):
                i = j * 2
                start(i + 1, 1)
                finish(i, 0)
                start(i + 2, 0)
                finish(i + 1, 1)

            finish(NCH2 - 1, 0)

        @pl.when(c == 0)
        def _():
            run(xsa_hbm)

        @pl.when(c == 1)
        def _():
            run(xsb_hbm)

        plsc.subcore_barrier()

        @pl.when(c == 0)
        def _():
            pltpu.sync_copy(acc_sh.at[sl], txa_hbm.at[sl])

            @pl.when(s == NS - 1)
            def _():
                pltpu.sync_copy(acc_sh.at[tl], txa_hbm.at[tl])

        @pl.when(c == 1)
        def _():
            pltpu.sync_copy(acc_sh.at[sl], txb_hbm.at[sl])

            @pl.when(s == NS - 1)
            def _():
                pltpu.sync_copy(acc_sh.at[tl], txb_hbm.at[tl])

    return k(row2, col2, xsa, xsb, zeros)


def _xw0_body(x_ref, w0_ref, b_ref, o_ref):
    o_ref[...] = jnp.dot(x_ref[...], w0_ref[...],
                         preferred_element_type=jnp.float32) + b_ref[...]


@jax.jit
def _xw0(x, W0, b2):
    grid = (N // _RB,)
    return pl.pallas_call(
        _xw0_body,
        grid=grid,
        in_specs=[
            pl.BlockSpec((_RB, D), lambda i: (i, 0)),
            pl.BlockSpec((D, D), lambda i: (0, 0)),
            pl.BlockSpec((1, D), lambda i: (0, 0)),
        ],
        out_specs=pl.BlockSpec((_RB, D), lambda i: (i, 0)),
        out_shape=jax.ShapeDtypeStruct((N, D), jnp.float32),
    )(x, W0, b2)


def _final_body(p_ref, ta_ref, tb_ref, da_ref, db_ref, w1_ref, o_ref):
    d1 = _dinv_block(da_ref[:, 0:1], db_ref[:, 0:1])
    acc = p_ref[...]
    acc += jnp.dot(ta_ref[...] * d1, w1_ref[:H, :],
                   preferred_element_type=jnp.float32)
    acc += jnp.dot(tb_ref[...] * d1, w1_ref[H:, :],
                   preferred_element_type=jnp.float32)
    o_ref[...] = jnp.maximum(acc, 0.0)


@jax.jit
def _final(p, txa, txb, dega, degb, W1):
    grid = (N // _RB,)
    return pl.pallas_call(
        _final_body,
        grid=grid,
        in_specs=[
            pl.BlockSpec((_RB, D), lambda i: (i, 0)),
            pl.BlockSpec((_RB, H), lambda i: (i, 0)),
            pl.BlockSpec((_RB, H), lambda i: (i, 0)),
            pl.BlockSpec((_RB, H), lambda i: (i, 0)),
            pl.BlockSpec((_RB, H), lambda i: (i, 0)),
            pl.BlockSpec((D, D), lambda i: (0, 0)),
        ],
        out_specs=pl.BlockSpec((_RB, D), lambda i: (i, 0)),
        out_shape=jax.ShapeDtypeStruct((N, D), jnp.float32),
    )(p, txa, txb, dega, degb, W1)


def kernel(x, edge_index, W0, W1, b):
    row2 = edge_index[0]
    col2 = edge_index[1]
    ones = jnp.ones((K1, H), jnp.float32)
    zeros = jnp.zeros((N, H), jnp.float32)
    p = _xw0(x, W0, b.reshape(1, D))
    dega, degb = _deg_sc(row2, ones, zeros)
    xsa, xsb = _prescale(x, dega, degb)
    txa, txb = _scatter_sc(row2, col2, xsa, xsb, zeros)
    return _final(p, txa, txb, dega, degb, W1)


# speedup vs baseline: 15.2364x; 1.2333x over previous
import functools

import jax
import jax.numpy as jnp
from jax import lax
from jax.experimental import pallas as pl
from jax.experimental.pallas import tpu as pltpu
from jax.experimental.pallas import tpu_sc as plsc

N = 10000
E = 160000
D = 256
H = 128
NC = 2
NS = 16
RA = 624
TAIL_OFF = RA * NS
TAIL = N - TAIL_OFF

EP1 = E // (NC * NS)
K1 = 40
NCH1 = EP1 // K1

EP2 = E // NS
K2 = 80
NCH2 = EP2 // K2

_mesh = plsc.VectorSubcoreMesh(core_axis_name="c", subcore_axis_name="s")


@jax.jit
def _deg_sc(row3, ones, zeros):
    @functools.partial(
        pl.kernel,
        out_type=[
            jax.ShapeDtypeStruct((N, H), jnp.float32),
            jax.ShapeDtypeStruct((N, H), jnp.float32),
        ],
        mesh=_mesh,
        scratch_types=[
            pltpu.VMEM_SHARED((N, H), jnp.float32),
            pltpu.VMEM((NCH1, K1), jnp.int32),
            pltpu.VMEM((K1, H), jnp.float32),
            pltpu.SemaphoreType.DMA,
        ],
    )
    def k(row3_hbm, ones_hbm, zeros_hbm, dega_hbm, degb_hbm,
          acc_sh, idx_v, ones_v, ssem):
        c = lax.axis_index("c")
        s = lax.axis_index("s")

        pltpu.sync_copy(ones_hbm, ones_v)
        sl = pl.ds(s * RA, RA)
        tl = pl.ds(TAIL_OFF, TAIL)
        pltpu.sync_copy(zeros_hbm.at[sl], acc_sh.at[sl])

        @pl.when(s == NS - 1)
        def _():
            pltpu.sync_copy(zeros_hbm.at[tl], acc_sh.at[tl])

        wid = s * NC + c
        pltpu.sync_copy(row3_hbm.at[wid], idx_v)
        plsc.subcore_barrier()

        @pl.loop(0, NCH1)
        def _(i):
            pltpu.async_copy(ones_v, acc_sh.at[idx_v.at[i]], ssem, add=True)

        @pl.loop(0, NCH1)
        def _(i):
            pltpu.make_async_copy(ones_v, acc_sh.at[idx_v.at[0]],
                                  ssem).wait()

        plsc.subcore_barrier()

        @pl.when(c == 0)
        def _():
            pltpu.sync_copy(acc_sh.at[sl], dega_hbm.at[sl])

            @pl.when(s == NS - 1)
            def _():
                pltpu.sync_copy(acc_sh.at[tl], dega_hbm.at[tl])

        @pl.when(c == 1)
        def _():
            pltpu.sync_copy(acc_sh.at[sl], degb_hbm.at[sl])

            @pl.when(s == NS - 1)
            def _():
                pltpu.sync_copy(acc_sh.at[tl], degb_hbm.at[tl])

    return k(row3, ones, zeros)


_RB = 1000


def _dinv_block(da, db):
    deg = da + db
    return jnp.where(deg > 0, lax.rsqrt(jnp.maximum(deg, 1.0)), 0.0)


def _prescale_body(x_ref, da_ref, db_ref, oa_ref, ob_ref):
    d1 = _dinv_block(da_ref[:, 0:1], db_ref[:, 0:1])
    xs = x_ref[...] * (-d1)
    oa_ref[...] = xs[:, :H]
    ob_ref[...] = xs[:, H:]


@jax.jit
def _prescale(x, dega, degb):
    grid = (N // _RB,)
    return pl.pallas_call(
        _prescale_body,
        grid=grid,
        in_specs=[
            pl.BlockSpec((_RB, D), lambda i: (i, 0)),
            pl.BlockSpec((_RB, H), lambda i: (i, 0)),
            pl.BlockSpec((_RB, H), lambda i: (i, 0)),
        ],
        out_specs=[
            pl.BlockSpec((_RB, H), lambda i: (i, 0)),
            pl.BlockSpec((_RB, H), lambda i: (i, 0)),
        ],
        out_shape=[
            jax.ShapeDtypeStruct((N, H), jnp.float32),
            jax.ShapeDtypeStruct((N, H), jnp.float32),
        ],
    )(x, dega, degb)


@jax.jit
def _scatter_sc(row2, col2, xsa, xsb, zeros):
    @functools.partial(
        pl.kernel,
        out_type=[
            jax.ShapeDtypeStruct((N, H), jnp.float32),
            jax.ShapeDtypeStruct((N, H), jnp.float32),
        ],
        mesh=_mesh,
        scratch_types=[
            pltpu.VMEM_SHARED((N, H), jnp.float32),
            pltpu.VMEM((EP2,), jnp.int32),
            pltpu.VMEM((K2,), jnp.int32),
            pltpu.VMEM((K2,), jnp.int32),
            pltpu.VMEM((K2, H), jnp.float32),
            pltpu.VMEM((K2, H), jnp.float32),
            pltpu.SemaphoreType.DMA,
            pltpu.SemaphoreType.DMA,
            pltpu.SemaphoreType.DMA,
            pltpu.SemaphoreType.DMA,
        ],
    )
    def k(row_hbm, col_hbm, xsa_hbm, xsb_hbm, zeros_hbm, txa_hbm, txb_hbm,
          acc_sh, ridx_v, cidx0_v, cidx1_v, rows0_v, rows1_v,
          gsem0, gsem1, csem0, csem1):
        c = lax.axis_index("c")
        s = lax.axis_index("s")

        sl = pl.ds(s * RA, RA)
        tl = pl.ds(TAIL_OFF, TAIL)
        pltpu.sync_copy(zeros_hbm.at[sl], acc_sh.at[sl])

        @pl.when(s == NS - 1)
        def _():
            pltpu.sync_copy(zeros_hbm.at[tl], acc_sh.at[tl])

        plsc.subcore_barrier()

        base = s * EP2
        cidx = (cidx0_v, cidx1_v)
        rows = (rows0_v, rows1_v)
        gsem = (gsem0, gsem1)
        csem = (csem0, csem1)

        def run(xs_hbm):
            pltpu.sync_copy(row_hbm.at[pl.ds(base, EP2)], ridx_v)

            def start(i, b):
                pltpu.async_copy(col_hbm.at[pl.ds(base + i * K2, K2)],
                                 cidx[b], csem[b])
                pltpu.async_copy(
                    xs_hbm.at[ridx_v.at[pl.ds(i * K2, K2)]], rows[b], gsem[b])

            def finish(i, b):
                pltpu.make_async_copy(col_hbm.at[pl.ds(base, K2)],
                                      cidx[b], csem[b]).wait()
                pltpu.make_async_copy(zeros_hbm.at[pl.ds(0, K2)], rows[b],
                                      gsem[b]).wait()
                pltpu.sync_copy(rows[b], acc_sh.at[cidx[b]], add=True)

            start(0, 0)

            @pl.loop(0, (NCH2 - 1) // 2)
            def _(j):
                i = j * 2
                start(i + 1, 1)
                finish(i, 0)
                start(i + 2, 0)
                finish(i + 1, 1)

            finish(NCH2 - 1, 0)

        @pl.when(c == 0)
        def _():
            run(xsa_hbm)

        @pl.when(c == 1)
        def _():
            run(xsb_hbm)

        plsc.subcore_barrier()

        @pl.when(c == 0)
        def _():
            pltpu.sync_copy(acc_sh.at[sl], txa_hbm.at[sl])

            @pl.when(s == NS - 1)
            def _():
                pltpu.sync_copy(acc_sh.at[tl], txa_hbm.at[tl])

        @pl.when(c == 1)
        def _():
            pltpu.sync_copy(acc_sh.at[sl], txb_hbm.at[sl])

            @pl.when(s == NS - 1)
            def _():
                pltpu.sync_copy(acc_sh.at[tl], txb_hbm.at[tl])

    return k(row2, col2, xsa, xsb, zeros)


def _xw0_body(x_ref, w0_ref, b_ref, o_ref):
    o_ref[...] = jnp.dot(x_ref[...], w0_ref[...],
                         preferred_element_type=jnp.float32) + b_ref[...]


@jax.jit
def _xw0(x, W0, b2):
    grid = (N // _RB,)
    return pl.pallas_call(
        _xw0_body,
        grid=grid,
        in_specs=[
            pl.BlockSpec((_RB, D), lambda i: (i, 0)),
            pl.BlockSpec((D, D), lambda i: (0, 0)),
            pl.BlockSpec((1, D), lambda i: (0, 0)),
        ],
        out_specs=pl.BlockSpec((_RB, D), lambda i: (i, 0)),
        out_shape=jax.ShapeDtypeStruct((N, D), jnp.float32),
    )(x, W0, b2)


def _final_body(p_ref, ta_ref, tb_ref, da_ref, db_ref, w1_ref, o_ref):
    d1 = _dinv_block(da_ref[:, 0:1], db_ref[:, 0:1])
    acc = p_ref[...]
    acc += jnp.dot(ta_ref[...] * d1, w1_ref[:H, :],
                   preferred_element_type=jnp.float32)
    acc += jnp.dot(tb_ref[...] * d1, w1_ref[H:, :],
                   preferred_element_type=jnp.float32)
    o_ref[...] = jnp.maximum(acc, 0.0)


@jax.jit
def _final(p, txa, txb, dega, degb, W1):
    grid = (N // _RB,)
    return pl.pallas_call(
        _final_body,
        grid=grid,
        in_specs=[
            pl.BlockSpec((_RB, D), lambda i: (i, 0)),
            pl.BlockSpec((_RB, H), lambda i: (i, 0)),
            pl.BlockSpec((_RB, H), lambda i: (i, 0)),
            pl.BlockSpec((_RB, H), lambda i: (i, 0)),
            pl.BlockSpec((_RB, H), lambda i: (i, 0)),
            pl.BlockSpec((D, D), lambda i: (0, 0)),
        ],
        out_specs=pl.BlockSpec((_RB, D), lambda i: (i, 0)),
        out_shape=jax.ShapeDtypeStruct((N, D), jnp.float32),
    )(p, txa, txb, dega, degb, W1)


def kernel(x, edge_index, W0, W1, b):
    row2 = edge_index[0]
    col2 = edge_index[1]
    row3 = row2.reshape(NC * NS, NCH1, K1)
    ones = jnp.ones((K1, H), jnp.float32)
    zeros = jnp.zeros((N, H), jnp.float32)
    p = _xw0(x, W0, b.reshape(1, D))
    dega, degb = _deg_sc(row3, ones, zeros)
    xsa, xsb = _prescale(x, dega, degb)
    txa, txb = _scatter_sc(row2, col2, xsa, xsb, zeros)
    return _final(p, txa, txb, dega, degb, W1)
